# baseline (speedup 1.0000x reference)
import jax
import jax.numpy as jnp
from jax import lax
from jax.experimental import pallas as pl
from jax.experimental.pallas import tpu as pltpu
from jax.experimental.pallas import tpu_sc as plsc

D_MODEL = 2048
SEQ_LEN = 2048
LANES = 16

_NC = 2
_NS = 16
_NW = _NC * _NS
_ROWS_PER_W = SEQ_LEN // _NW
_CHUNK = 16
_NCHUNK = _ROWS_PER_W // _CHUNK

_mesh = plsc.VectorSubcoreMesh(
    core_axis_name="c", subcore_axis_name="s",
    num_cores=_NC, num_subcores=_NS)


def _emb_add_body(x_hbm, table_hbm, out_hbm,
                  xi_v, xf_v, buf0, buf1, buf2,
                  gsem0, gsem1, gsem2, osem0, osem1, osem2):
    wid = lax.axis_index("s") * _NC + lax.axis_index("c")
    base = wid * _ROWS_PER_W

    pltpu.sync_copy(x_hbm, xi_v)
    bufs = (buf0, buf1, buf2)
    gsems = (gsem0, gsem1, gsem2)
    osems = (osem0, osem1, osem2)
    pltpu.async_copy(table_hbm.at[xi_v[pl.ds(base, _CHUNK)]], bufs[0], gsems[0])

    @plsc.parallel_loop(0, SEQ_LEN // LANES)
    def _cvt(j):
        xf_v[pl.ds(j * LANES, LANES)] = (
            xi_v[pl.ds(j * LANES, LANES)].astype(jnp.float32))

    for g in range(_NCHUNK):
        b = g % 3
        pltpu.make_async_copy(
            table_hbm.at[xi_v[pl.ds(base + g * _CHUNK, _CHUNK)]],
            bufs[b], gsems[b]).wait()
        if g + 1 < _NCHUNK:
            nb = (g + 1) % 3
            if g >= 2:
                pltpu.make_async_copy(
                    bufs[nb],
                    out_hbm.at[pl.ds(base + (g - 2) * _CHUNK, _CHUNK)],
                    osems[nb],
                ).wait()
            pltpu.async_copy(
                table_hbm.at[xi_v[pl.ds(base + (g + 1) * _CHUNK, _CHUNK)]],
                bufs[nb], gsems[nb])

        for h in range(2):
            @plsc.parallel_loop(0, D_MODEL // LANES)
            def _add_row_vec(j, _b=b, _h=h):
                xv = xf_v[pl.ds(j * LANES, LANES)]
                for r in range(_h * (_CHUNK // 2), (_h + 1) * (_CHUNK // 2)):
                    plsc.addupdate(bufs[_b].at[r, pl.ds(j * LANES, LANES)], xv)

            pltpu.async_copy(
                bufs[b].at[pl.ds(h * (_CHUNK // 2), _CHUNK // 2)],
                out_hbm.at[pl.ds(base + g * _CHUNK + h * (_CHUNK // 2),
                                 _CHUNK // 2)],
                osems[b])

    for gd in range(max(0, _NCHUNK - 3), _NCHUNK):
        pltpu.make_async_copy(
            bufs[gd % 3], out_hbm.at[pl.ds(base + gd * _CHUNK, _CHUNK)],
            osems[gd % 3]).wait()


_SCRATCH = [
    pltpu.VMEM((SEQ_LEN,), jnp.int32),
    pltpu.VMEM((D_MODEL,), jnp.float32),
    pltpu.VMEM((_CHUNK, D_MODEL), jnp.float32),
    pltpu.VMEM((_CHUNK, D_MODEL), jnp.float32),
    pltpu.VMEM((_CHUNK, D_MODEL), jnp.float32),
    pltpu.SemaphoreType.DMA,
    pltpu.SemaphoreType.DMA,
    pltpu.SemaphoreType.DMA,
    pltpu.SemaphoreType.DMA,
    pltpu.SemaphoreType.DMA,
    pltpu.SemaphoreType.DMA,
]

_emb_add = pl.kernel(
    _emb_add_body,
    out_type=jax.ShapeDtypeStruct((SEQ_LEN, D_MODEL), jnp.float32),
    mesh=_mesh,
    scratch_types=_SCRATCH,
)


def kernel(x, embedding_weight):
    return _emb_add(x, embedding_weight)


# speedup vs baseline: 1.0049x; 1.0049x over previous
import jax
import jax.numpy as jnp
from jax import lax
from jax.experimental import pallas as pl
from jax.experimental.pallas import tpu as pltpu
from jax.experimental.pallas import tpu_sc as plsc

D_MODEL = 2048
SEQ_LEN = 2048
LANES = 16

_NC = 2
_NS = 16
_NW = _NC * _NS
_ROWS_PER_W = SEQ_LEN // _NW
_CHUNK = 16
_NCHUNK = _ROWS_PER_W // _CHUNK

_mesh = plsc.VectorSubcoreMesh(
    core_axis_name="c", subcore_axis_name="s",
    num_cores=_NC, num_subcores=_NS)


def _emb_add_body(x_hbm, table_hbm, out_hbm,
                  xi_v, xf_v, buf0, buf1, gsem0, gsem1, osem0, osem1):
    wid = lax.axis_index("s") * _NC + lax.axis_index("c")
    base = wid * _ROWS_PER_W

    pltpu.sync_copy(x_hbm, xi_v)
    bufs = (buf0, buf1)
    gsems = (gsem0, gsem1)
    osems = (osem0, osem1)
    pltpu.async_copy(table_hbm.at[xi_v[pl.ds(base, _CHUNK)]], bufs[0], gsems[0])

    @plsc.parallel_loop(0, SEQ_LEN // LANES)
    def _cvt(j):
        xf_v[pl.ds(j * LANES, LANES)] = (
            xi_v[pl.ds(j * LANES, LANES)].astype(jnp.float32))

    for g in range(_NCHUNK):
        b = g & 1
        pltpu.make_async_copy(
            table_hbm.at[xi_v[pl.ds(base + g * _CHUNK, _CHUNK)]],
            bufs[b], gsems[b]).wait()
        if g + 1 < _NCHUNK:
            nb = (g + 1) & 1
            if g >= 1:
                pltpu.make_async_copy(
                    bufs[nb],
                    out_hbm.at[pl.ds(base + (g - 1) * _CHUNK, _CHUNK)],
                    osems[nb],
                ).wait()
            pltpu.async_copy(
                table_hbm.at[xi_v[pl.ds(base + (g + 1) * _CHUNK, _CHUNK)]],
                bufs[nb], gsems[nb])

        for h in range(2):
            @plsc.parallel_loop(0, D_MODEL // LANES)
            def _add_row_vec(j, _b=b, _h=h):
                xv = xf_v[pl.ds(j * LANES, LANES)]
                for r in range(_h * (_CHUNK // 2), (_h + 1) * (_CHUNK // 2)):
                    plsc.addupdate(bufs[_b].at[r, pl.ds(j * LANES, LANES)], xv)

            pltpu.async_copy(
                bufs[b].at[pl.ds(h * (_CHUNK // 2), _CHUNK // 2)],
                out_hbm.at[pl.ds(base + g * _CHUNK + h * (_CHUNK // 2),
                                 _CHUNK // 2)],
                osems[b])

    pltpu.make_async_copy(
        bufs[0], out_hbm.at[pl.ds(base + (_NCHUNK - 2) * _CHUNK, _CHUNK)],
        osems[0]).wait()
    pltpu.make_async_copy(
        bufs[1], out_hbm.at[pl.ds(base + (_NCHUNK - 1) * _CHUNK, _CHUNK)],
        osems[1]).wait()


_SCRATCH = [
    pltpu.VMEM((SEQ_LEN,), jnp.int32),
    pltpu.VMEM((D_MODEL,), jnp.float32),
    pltpu.VMEM((_CHUNK, D_MODEL), jnp.float32),
    pltpu.VMEM((_CHUNK, D_MODEL), jnp.float32),
    pltpu.SemaphoreType.DMA,
    pltpu.SemaphoreType.DMA,
    pltpu.SemaphoreType.DMA,
    pltpu.SemaphoreType.DMA,
]

_emb_add = pl.kernel(
    _emb_add_body,
    out_type=jax.ShapeDtypeStruct((SEQ_LEN, D_MODEL), jnp.float32),
    mesh=_mesh,
    scratch_types=_SCRATCH,
)


def kernel(x, embedding_weight):
    return _emb_add(x, embedding_weight)
